# baseline (speedup 1.0000x reference)
import jax
import jax.numpy as jnp
from jax import lax
from jax.experimental import pallas as pl
from jax.experimental.pallas import tpu as pltpu
from jax.experimental.pallas import tpu_sc as plsc

SEQ = 8192
DM = 1024

_info = plsc.get_sparse_core_info()
_NC = _info.num_cores
_NS = _info.num_subcores
_L = _info.num_lanes
_NW = _NC * _NS
_RPW = SEQ // _NW
_CHUNK = 8
_NSTEP = _RPW // _CHUNK
_NBT = 3
_NBS = 3
_XA = 1
_VPR = DM // _L
_SPR = _NS * _NBS * _CHUNK


def _body(x_hbm, table_hbm, pe_hbm, out_hbm, *scratch):
    xb = scratch[0:_NBT]
    tb = scratch[_NBT:2 * _NBT]
    idxb = scratch[2 * _NBT]
    shared = scratch[2 * _NBT + 1]
    sems = scratch[2 * _NBT + 2:]
    semxs = sems[0:_NBS]
    semxv = sems[_NBS:_NBS + _NBT]
    semt = sems[_NBS + _NBT:_NBS + 2 * _NBT]
    semw = sems[_NBS + 2 * _NBT:2 * _NBS + 2 * _NBT]
    semo = sems[2 * _NBS + 2 * _NBT:3 * _NBS + 2 * _NBT]

    cid = lax.axis_index("c")
    sid = lax.axis_index("s")
    wid = sid * _NC + cid
    base = wid * _RPW
    sp0 = sid * (_NBS * _CHUNK)

    pltpu.sync_copy(pe_hbm.at[pl.ds(base, _RPW)], idxb)

    def spslot(i):
        return pl.ds(sp0 + (i % _NBS) * _CHUNK, _CHUNK)

    def issue_xs(i):
        row = base + i * _CHUNK
        return pltpu.async_copy(
            x_hbm.at[pl.ds(row, _CHUNK)], shared.at[spslot(i)],
            semxs[i % _NBS])

    def issue_xv(i):
        return pltpu.async_copy(
            shared.at[spslot(i)], xb[i % _NBT], semxv[i % _NBT])

    def issue_t(i):
        return pltpu.async_copy(
            table_hbm.at[idxb.at[pl.ds(i * _CHUNK, _CHUNK)]],
            tb[i % _NBT], semt[i % _NBT])

    pxs, pxv, pt, pw, po = {}, {}, {}, {}, {}
    for j in range(min(_XA, _NSTEP)):
        pxs[j] = issue_xs(j)
    for j in range(min(_NBT - 1, _NSTEP)):
        pt[j] = issue_t(j)

    for i in range(_NSTEP):
        if i - 1 in pw:
            pw.pop(i - 1).wait()
            po[i - 1] = pltpu.async_copy(
                shared.at[spslot(i - 1)],
                out_hbm.at[pl.ds(base + (i - 1) * _CHUNK, _CHUNK)],
                semo[(i - 1) % _NBS])
        nxt = i + _XA
        if nxt < _NSTEP:
            prev = nxt - _NBS
            if prev in po:
                po.pop(prev).wait()
            pxs[nxt] = issue_xs(nxt)
        pxs.pop(i).wait()
        pxv[i] = issue_xv(i)
        tn = i + _NBT - 1
        if tn < _NSTEP:
            pt[tn] = issue_t(tn)

        pxv.pop(i).wait()
        pt.pop(i).wait()
        b = i % _NBT
        xb_b, tb_b = xb[b], tb[b]

        @plsc.parallel_loop(0, _CHUNK * _VPR, step=1, unroll=8)
        def compute(j, xb_b=xb_b, tb_b=tb_b):
            r = lax.shift_right_logical(j, 6)
            c = pl.multiple_of(
                lax.shift_left(lax.bitwise_and(j, _VPR - 1), 4), _L)
            sl = pl.ds(c, _L)
            plsc.addupdate(tb_b.at[r, sl], xb_b[r, sl])

        pw[i] = pltpu.async_copy(tb_b, shared.at[spslot(i)], semw[i % _NBS])

    i = _NSTEP - 1
    pw.pop(i).wait()
    po[i] = pltpu.async_copy(
        shared.at[spslot(i)], out_hbm.at[pl.ds(base + i * _CHUNK, _CHUNK)],
        semo[i % _NBS])
    for i in sorted(po):
        po.pop(i).wait()


_pe_call = pl.kernel(
    _body,
    out_type=jax.ShapeDtypeStruct((SEQ, DM), jnp.float32),
    mesh=plsc.VectorSubcoreMesh(core_axis_name="c", subcore_axis_name="s"),
    scratch_types=(
        [pltpu.VMEM((_CHUNK, DM), jnp.float32) for _ in range(2 * _NBT)]
        + [pltpu.VMEM((_RPW,), jnp.int32)]
        + [pltpu.VMEM_SHARED((_SPR, DM), jnp.float32)]
        + [pltpu.SemaphoreType.DMA for _ in range(3 * _NBS + 2 * _NBT)]
    ),
)


@jax.jit
def kernel(x, table, pe):
    return _pe_call(x, table, pe)


# speedup vs baseline: 1.3224x; 1.3224x over previous
import jax
import jax.numpy as jnp
from jax import lax
from jax.experimental import pallas as pl
from jax.experimental.pallas import tpu as pltpu
from jax.experimental.pallas import tpu_sc as plsc

SEQ = 8192
DM = 1024

_info = plsc.get_sparse_core_info()
_NC = _info.num_cores
_NS = _info.num_subcores
_L = _info.num_lanes
_NW = _NC * _NS
_RPW = SEQ // _NW
_CHUNK = 8
_NSTEP = _RPW // _CHUNK
_NBUF = 6
_VPR = DM // _L


def _body(x_hbm, table_hbm, pe_hbm, out_hbm, *scratch):
    xb = scratch[0:_NBUF]
    tb = scratch[_NBUF:2 * _NBUF]
    idxb = scratch[2 * _NBUF]
    semx = scratch[2 * _NBUF + 1:2 * _NBUF + 1 + _NBUF]
    semt = scratch[2 * _NBUF + 1 + _NBUF:2 * _NBUF + 1 + 2 * _NBUF]
    semo = scratch[2 * _NBUF + 1 + 2 * _NBUF:2 * _NBUF + 1 + 3 * _NBUF]

    wid = lax.axis_index("s") * _NC + lax.axis_index("c")
    base = wid * _RPW
    pltpu.sync_copy(pe_hbm.at[pl.ds(base, _RPW)], idxb)

    def issue_in(i):
        b = i % _NBUF
        row = base + i * _CHUNK
        cx = pltpu.async_copy(x_hbm.at[pl.ds(row, _CHUNK)], xb[b], semx[b])
        ct = pltpu.async_copy(
            table_hbm.at[idxb.at[pl.ds(i * _CHUNK, _CHUNK)]], tb[b], semt[b])
        return cx, ct

    pending_in = {}
    pending_out = {}
    for j in range(_NBUF - 1):
        if j < _NSTEP:
            pending_in[j] = issue_in(j)

    for i in range(_NSTEP):
        b = i % _NBUF
        nxt = i + _NBUF - 1
        if nxt < _NSTEP:
            prev = nxt - _NBUF
            if prev >= 0:
                pending_out.pop(prev).wait()
            pending_in[nxt] = issue_in(nxt)
        cx, ct = pending_in.pop(i)
        cx.wait()
        ct.wait()

        xb_b, tb_b = xb[b], tb[b]

        @plsc.parallel_loop(0, _CHUNK * _VPR, step=1, unroll=8)
        def compute(j, xb_b=xb_b, tb_b=tb_b):
            r = lax.shift_right_logical(j, 6)
            c = pl.multiple_of(
                lax.shift_left(lax.bitwise_and(j, _VPR - 1), 4), _L)
            sl = pl.ds(c, _L)
            plsc.addupdate(tb_b.at[r, sl], xb_b[r, sl])
        row = base + i * _CHUNK
        pending_out[i] = pltpu.async_copy(
            tb_b, out_hbm.at[pl.ds(row, _CHUNK)], semo[b])

    for i in sorted(pending_out):
        pending_out.pop(i).wait()


_pe_call = pl.kernel(
    _body,
    out_type=jax.ShapeDtypeStruct((SEQ, DM), jnp.float32),
    mesh=plsc.VectorSubcoreMesh(core_axis_name="c", subcore_axis_name="s"),
    scratch_types=(
        [pltpu.VMEM((_CHUNK, DM), jnp.float32) for _ in range(2 * _NBUF)]
        + [pltpu.VMEM((_RPW,), jnp.int32)]
        + [pltpu.SemaphoreType.DMA for _ in range(3 * _NBUF)]
    ),
)


@jax.jit
def kernel(x, table, pe):
    return _pe_call(x, table, pe)


# speedup vs baseline: 1.3588x; 1.0275x over previous
import jax
import jax.numpy as jnp
from jax import lax
from jax.experimental import pallas as pl
from jax.experimental.pallas import tpu as pltpu
from jax.experimental.pallas import tpu_sc as plsc

SEQ = 8192
DM = 1024

_info = plsc.get_sparse_core_info()
_NC = _info.num_cores
_NS = _info.num_subcores
_L = _info.num_lanes
_NW = _NC * _NS
_RPW = SEQ // _NW
_CHUNK = 16
_NSTEP = _RPW // _CHUNK
_NBUF = 3
_VPR = DM // _L


def _body(x_hbm, table_hbm, pe_hbm, out_hbm, *scratch):
    xb = scratch[0:_NBUF]
    tb = scratch[_NBUF:2 * _NBUF]
    idxb = scratch[2 * _NBUF]
    semx = scratch[2 * _NBUF + 1:2 * _NBUF + 1 + _NBUF]
    semt = scratch[2 * _NBUF + 1 + _NBUF:2 * _NBUF + 1 + 2 * _NBUF]
    semo = scratch[2 * _NBUF + 1 + 2 * _NBUF:2 * _NBUF + 1 + 3 * _NBUF]
    sempe = scratch[2 * _NBUF + 1 + 3 * _NBUF]

    wid = lax.axis_index("s") * _NC + lax.axis_index("c")
    base = wid * _RPW

    def issue_x(i):
        b = i % _NBUF
        row = base + i * _CHUNK
        return pltpu.async_copy(x_hbm.at[pl.ds(row, _CHUNK)], xb[b], semx[b])

    def issue_t(i):
        b = i % _NBUF
        return pltpu.async_copy(
            table_hbm.at[idxb.at[pl.ds(i * _CHUNK, _CHUNK)]], tb[b], semt[b])

    def issue_in(i):
        return issue_x(i), issue_t(i)

    pe_cp = pltpu.async_copy(pe_hbm.at[pl.ds(base, _RPW)], idxb, sempe)
    first = [issue_x(j) for j in range(min(_NBUF - 1, _NSTEP))]
    pe_cp.wait()

    pending_in = {}
    pending_out = {}
    for j in range(min(_NBUF - 1, _NSTEP)):
        pending_in[j] = (first[j], issue_t(j))

    for i in range(_NSTEP):
        b = i % _NBUF
        nxt = i + _NBUF - 1
        if nxt < _NSTEP:
            prev = nxt - _NBUF
            if prev >= 0:
                pending_out.pop(prev).wait()
            pending_in[nxt] = issue_in(nxt)
        cx, ct = pending_in.pop(i)
        cx.wait()
        ct.wait()

        xb_b, tb_b = xb[b], tb[b]

        @plsc.parallel_loop(0, _CHUNK * _VPR, step=1, unroll=8)
        def compute(j, xb_b=xb_b, tb_b=tb_b):
            r = lax.shift_right_logical(j, 6)
            c = pl.multiple_of(
                lax.shift_left(lax.bitwise_and(j, _VPR - 1), 4), _L)
            sl = pl.ds(c, _L)
            plsc.addupdate(tb_b.at[r, sl], xb_b[r, sl])
        row = base + i * _CHUNK
        pending_out[i] = pltpu.async_copy(
            tb_b, out_hbm.at[pl.ds(row, _CHUNK)], semo[b])

    for i in sorted(pending_out):
        pending_out.pop(i).wait()


_pe_call = pl.kernel(
    _body,
    out_type=jax.ShapeDtypeStruct((SEQ, DM), jnp.float32),
    mesh=plsc.VectorSubcoreMesh(core_axis_name="c", subcore_axis_name="s"),
    scratch_types=(
        [pltpu.VMEM((_CHUNK, DM), jnp.float32) for _ in range(2 * _NBUF)]
        + [pltpu.VMEM((_RPW,), jnp.int32)]
        + [pltpu.SemaphoreType.DMA for _ in range(3 * _NBUF + 1)]
    ),
)


@jax.jit
def kernel(x, table, pe):
    return _pe_call(x, table, pe)
